# baseline (speedup 1.0000x reference)
import functools

import jax
import jax.numpy as jnp
from jax import lax
from jax.experimental import pallas as pl
from jax.experimental.pallas import tpu as pltpu
from jax.experimental.pallas import tpu_sc as plsc

N = 10000
E = 160000
IN_DIM = 256
HID_DIM = 512
OUT_DIM = 256

NC = 2
NS = 16
LANES = 16
HALF = 128
G = 128
NB = 80
E_PAD = NS * NB * G

GA = 64
RING = 4
NBA = E_PAD // (NS * GA)
CHA = 40
NCHA = NBA // CHA
NGEN = CHA // RING
N_ACC = 10112
ROWS_PER_SUB = N_ACC // NS
RB = 400
GRID = N // RB

_mesh = plsc.VectorSubcoreMesh(core_axis_name="c", subcore_axis_name="s")


def _const_fill(buf, rows, cols, value):
    vec = jnp.full((LANES,), value, jnp.float32)

    @pl.loop(0, rows)
    def _(r):
        @pl.loop(0, cols // LANES)
        def _(c):
            buf[r, pl.ds(c * LANES, LANES)] = vec


def _zero_fill(buf, rows, cols):
    _const_fill(buf, rows, cols, 0.0)


NBD = E_PAD // (NC * NS * G)


@functools.partial(
    pl.kernel,
    mesh=_mesh,
    out_type=jax.ShapeDtypeStruct((NC * N_ACC, HALF), jnp.float32),
    scratch_types=[
        pltpu.VMEM((NBD, G), jnp.int32),
        pltpu.VMEM((G, HALF), jnp.float32),
        pltpu.VMEM_SHARED((N_ACC, HALF), jnp.float32),
    ],
)
def _sc_degree(dst_hbm, out_hbm, idx_v, ones_v, acc_sh):
    cid = lax.axis_index("c")
    sid = lax.axis_index("s")

    pltpu.sync_copy(dst_hbm.at[cid * NS + sid], idx_v)

    _zero_fill(ones_v, G, HALF)
    r0 = sid * ROWS_PER_SUB

    @pl.loop(0, 4)
    def _(i):
        pltpu.sync_copy(ones_v, acc_sh.at[pl.ds(r0 + i * G, G)])

    pltpu.sync_copy(ones_v.at[pl.ds(0, ROWS_PER_SUB - 4 * G)],
                    acc_sh.at[pl.ds(r0 + 4 * G, ROWS_PER_SUB - 4 * G)])

    _const_fill(ones_v, G, HALF, 1.0)

    plsc.subcore_barrier()

    @pl.loop(0, NBD)
    def _(j):
        pltpu.sync_copy(ones_v, acc_sh.at[idx_v.at[j]], add=True)

    plsc.subcore_barrier()

    pltpu.sync_copy(acc_sh.at[pl.ds(r0, ROWS_PER_SUB)],
                    out_hbm.at[pl.ds(cid * N_ACC + r0, ROWS_PER_SUB)])


@functools.partial(
    pl.kernel,
    mesh=_mesh,
    out_type=jax.ShapeDtypeStruct((NC * N_ACC, HALF), jnp.float32),
    scratch_types=[
        pltpu.VMEM((CHA, GA), jnp.int32),
        pltpu.VMEM((CHA, GA), jnp.int32),
        pltpu.VMEM((RING, GA, HALF), jnp.float32),
        pltpu.VMEM_SHARED((N_ACC, HALF), jnp.float32),
    ] + [pltpu.SemaphoreType.DMA] * (2 * RING),
)
def _sc_aggregate(src_hbm, dst_hbm, table_hbm, out_hbm,
                  src_v, dst_v, bufs, acc_sh, *sems):
    gsems = sems[:RING]
    ssems = sems[RING:]
    cid = lax.axis_index("c")
    sid = lax.axis_index("s")

    _zero_fill(bufs.at[0], GA, HALF)
    r0 = sid * ROWS_PER_SUB
    nz = ROWS_PER_SUB // GA

    @pl.loop(0, nz)
    def _(i):
        pltpu.sync_copy(bufs.at[0], acc_sh.at[pl.ds(r0 + i * GA, GA)])

    pltpu.sync_copy(bufs.at[0].at[pl.ds(0, ROWS_PER_SUB - nz * GA)],
                    acc_sh.at[pl.ds(r0 + nz * GA, ROWS_PER_SUB - nz * GA)])

    off = cid * N
    plsc.subcore_barrier()

    @pl.loop(0, NCHA)
    def _(q):
        pltpu.sync_copy(src_hbm.at[sid, pl.ds(q * CHA, CHA)], src_v)
        pltpu.sync_copy(dst_hbm.at[sid, pl.ds(q * CHA, CHA)], dst_v)

        @pl.loop(0, CHA)
        def _(j):
            @pl.loop(0, GA // LANES)
            def _(c):
                src_v[j, pl.ds(c * LANES, LANES)] = (
                    src_v[j, pl.ds(c * LANES, LANES)] + off)

        for b in range(RING):
            pltpu.async_copy(table_hbm.at[src_v.at[b]], bufs.at[b], gsems[b])

        @pl.loop(0, NGEN)
        def _(i):
            j0 = i * RING
            for b in range(RING):
                pltpu.make_async_copy(table_hbm.at[src_v.at[0]],
                                      bufs.at[b], gsems[b]).wait()
                pltpu.async_copy(bufs.at[b], acc_sh.at[dst_v.at[j0 + b]],
                                 ssems[b], add=True)
            for b in range(RING):
                pltpu.make_async_copy(bufs.at[b], acc_sh.at[dst_v.at[0]],
                                      ssems[b]).wait()

                @pl.when(i < NGEN - 1)
                def _():
                    pltpu.async_copy(table_hbm.at[src_v.at[j0 + RING + b]],
                                     bufs.at[b], gsems[b])

    plsc.subcore_barrier()

    pltpu.sync_copy(acc_sh.at[pl.ds(r0, ROWS_PER_SUB)],
                    out_hbm.at[pl.ds(cid * N_ACC + r0, ROWS_PER_SUB)])


def _dinv_of(deg_ref):
    return lax.rsqrt(deg_ref[0][:, :1] + deg_ref[1][:, :1] + 1.0)


def _scale_split(deg_ref, x_ref, o_ref):
    xp = x_ref[...] * _dinv_of(deg_ref)
    o_ref[0] = xp[:, :HALF]
    o_ref[1] = xp[:, HALF:]


def _mm_chain(agg_ref, xp_ref, deg_ref, w1_ref, b1_ref, w2_ref,
              h_ref, zp_ref):
    dinv = _dinv_of(deg_ref)
    y = jnp.concatenate([(agg_ref[0] + xp_ref[0]) * dinv,
                         (agg_ref[1] + xp_ref[1]) * dinv], axis=1)
    x1 = jnp.dot(y, w1_ref[...], preferred_element_type=jnp.float32)
    h = jnp.maximum(x1 + b1_ref[...], 0.0)
    h_ref[...] = h
    z = jnp.dot(h, w2_ref[...], preferred_element_type=jnp.float32)
    zp = z * dinv
    zp_ref[0] = zp[:, :HALF]
    zp_ref[1] = zp[:, HALF:]


def _merge_bias(agg_ref, zp_ref, deg_ref, b2_ref, o_ref):
    dinv = _dinv_of(deg_ref)
    o_ref[...] = jnp.concatenate([(agg_ref[0] + zp_ref[0]) * dinv,
                                  (agg_ref[1] + zp_ref[1]) * dinv],
                                 axis=1) + b2_ref[...]


def kernel(x, edge_index, W1, b1, W2, b2):
    src = edge_index[0]
    dst = edge_index[1]
    pad = E_PAD - E
    srcp = jnp.concatenate([src, jnp.zeros((pad,), jnp.int32)]).reshape(NS, NB, G)
    dstp = jnp.concatenate([dst, jnp.full((pad,), N, jnp.int32)]).reshape(NS, NB, G)

    deg2 = _sc_degree(dstp.reshape(NC * NS, NBD, G)).reshape(NC, N_ACC, HALF)

    xp_st = pl.pallas_call(
        _scale_split,
        grid=(GRID,),
        in_specs=[pl.BlockSpec((2, RB, HALF), lambda i: (0, i, 0)),
                  pl.BlockSpec((RB, IN_DIM), lambda i: (i, 0))],
        out_specs=pl.BlockSpec((2, RB, HALF), lambda i: (0, i, 0)),
        out_shape=jax.ShapeDtypeStruct((2, N, HALF), jnp.float32),
    )(deg2, x)

    srcpa = srcp.reshape(NS, NBA, GA)
    dstpa = dstp.reshape(NS, NBA, GA)
    agg1 = _sc_aggregate(srcpa, dstpa, xp_st.reshape(2 * N, HALF))
    agg1 = agg1.reshape(2, N_ACC, HALF)

    h, zp_st = pl.pallas_call(
        _mm_chain,
        grid=(GRID,),
        in_specs=[pl.BlockSpec((2, RB, HALF), lambda i: (0, i, 0)),
                  pl.BlockSpec((2, RB, HALF), lambda i: (0, i, 0)),
                  pl.BlockSpec((2, RB, HALF), lambda i: (0, i, 0)),
                  pl.BlockSpec((IN_DIM, HID_DIM), lambda i: (0, 0)),
                  pl.BlockSpec((1, HID_DIM), lambda i: (0, 0)),
                  pl.BlockSpec((HID_DIM, OUT_DIM), lambda i: (0, 0))],
        out_specs=[pl.BlockSpec((RB, HID_DIM), lambda i: (i, 0)),
                   pl.BlockSpec((2, RB, HALF), lambda i: (0, i, 0))],
        out_shape=[jax.ShapeDtypeStruct((N, HID_DIM), jnp.float32),
                   jax.ShapeDtypeStruct((2, N, HALF), jnp.float32)],
    )(agg1, xp_st, deg2, W1, b1.reshape(1, HID_DIM), W2)

    agg2 = _sc_aggregate(srcpa, dstpa, zp_st.reshape(2 * N, HALF))
    agg2 = agg2.reshape(2, N_ACC, HALF)

    x2 = pl.pallas_call(
        _merge_bias,
        grid=(GRID,),
        in_specs=[pl.BlockSpec((2, RB, HALF), lambda i: (0, i, 0)),
                  pl.BlockSpec((2, RB, HALF), lambda i: (0, i, 0)),
                  pl.BlockSpec((2, RB, HALF), lambda i: (0, i, 0)),
                  pl.BlockSpec((1, OUT_DIM), lambda i: (0, 0))],
        out_specs=pl.BlockSpec((RB, OUT_DIM), lambda i: (i, 0)),
        out_shape=jax.ShapeDtypeStruct((N, OUT_DIM), jnp.float32),
    )(agg2, zp_st, deg2, b2.reshape(1, OUT_DIM))

    return (x2, h)
